# baseline (speedup 1.0000x reference)
import functools

import jax
import jax.numpy as jnp
import numpy as np
from jax import lax
from jax.experimental import pallas as pl
from jax.experimental.pallas import tpu as pltpu
from jax.experimental.pallas import tpu_sc as plsc

FIELD_DIMS = [100000] * 26
EMBED_DIM = 16
BATCH = 4096
NUM_FIELDS = len(FIELD_DIMS)
TOTAL = sum(FIELD_DIMS)

NC, NS, L = 2, 16, 16
NW = NC * NS
B_PER_W = BATCH // NW
ROWS_PER_W = B_PER_W * NUM_FIELDS
GROUPS = B_PER_W // L
GROW = L * NUM_FIELDS
GCHUNK = 104
NGC = GROW // GCHUNK
LCHUNK = 128
NLC = ROWS_PER_W // LCHUNK


def _fm_body(blk_hbm, off_hbm, lidx_hbm, bias_hbm, emb_hbm, lin_hbm, out_hbm,
             blkidx_v, off_v, lidx_v, lin_v, buf0, out_v, bias_v,
             sem_e, sem_l):
  wid = lax.axis_index("s") * NC + lax.axis_index("c")

  pltpu.sync_copy(blk_hbm.at[wid], blkidx_v)
  pltpu.sync_copy(off_hbm.at[wid], off_v)
  pltpu.sync_copy(lidx_hbm.at[wid], lidx_v)
  pltpu.sync_copy(bias_hbm, bias_v)

  def fire_group(g, buf):
    for c in range(NGC):
      pltpu.make_async_copy(
          emb_hbm.at[blkidx_v.at[pl.ds(g * GROW + c * GCHUNK, GCHUNK)]],
          buf.at[pl.ds(c * GCHUNK, GCHUNK)],
          sem_e,
      ).start()

  def wait_group(buf):
    pltpu.make_async_copy(emb_hbm.at[pl.ds(0, GROW)], buf, sem_e).wait()

  def fire_lin(j, _):
    pltpu.make_async_copy(
        lin_hbm.at[lidx_v.at[j]], lin_v.at[pl.ds(j * LCHUNK, LCHUNK)], sem_l
    ).start()
    return _

  fire_group(0, buf0)
  lax.fori_loop(0, NLC, fire_lin, None)
  pltpu.make_async_copy(lin_hbm.at[pl.ds(0, ROWS_PER_W)], lin_v, sem_l).wait()

  lane = lax.iota(jnp.int32, L)
  rowv = [lane * NUM_FIELDS + f for f in range(NUM_FIELDS)]
  bias_vec = bias_v[:]

  def compute_group(g, buf):
    acc = bias_vec
    ssq = jnp.zeros((L,), jnp.float32)
    sd = [jnp.zeros((L,), jnp.float32) for _ in range(EMBED_DIM)]
    for f in range(NUM_FIELDS):
      src = pl.ds(f * B_PER_W + g * L, L)
      acc = acc + lin_v[src]
      off = off_v[src]
      for d in range(EMBED_DIM):
        v = plsc.load_gather(buf, [rowv[f], off + d])
        sd[d] = sd[d] + v
        ssq = ssq + v * v

    s2 = jnp.zeros((L,), jnp.float32)
    for d in range(EMBED_DIM):
      s2 = s2 + sd[d] * sd[d]

    out_v[pl.ds(g * L, L)] = acc + 0.5 * (s2 - ssq)

  def step(g, carry):
    wait_group(buf0)
    compute_group(g, buf0)

    @pl.when(g < GROUPS - 1)
    def _fire0():
      fire_group(g + 1, buf0)

    return carry

  lax.fori_loop(0, GROUPS, step, None)

  pltpu.sync_copy(out_v, out_hbm.at[pl.ds(wid * B_PER_W, B_PER_W)])


_fm_call = functools.partial(
    pl.kernel,
    out_type=jax.ShapeDtypeStruct((BATCH,), jnp.float32),
    mesh=plsc.VectorSubcoreMesh(core_axis_name="c", subcore_axis_name="s"),
    compiler_params=pltpu.CompilerParams(needs_layout_passes=False),
    scratch_types=[
        pltpu.VMEM((ROWS_PER_W,), jnp.int32),
        pltpu.VMEM((ROWS_PER_W,), jnp.int32),
        pltpu.VMEM((NLC, LCHUNK), jnp.int32),
        pltpu.VMEM((ROWS_PER_W,), jnp.float32),
        pltpu.VMEM((GROW, 128), jnp.float32),
        pltpu.VMEM((B_PER_W,), jnp.float32),
        pltpu.VMEM((L,), jnp.float32),
        pltpu.SemaphoreType.DMA,
        pltpu.SemaphoreType.DMA,
    ],
)(_fm_body)

_OFFSETS = np.concatenate([[0], np.cumsum(FIELD_DIMS)[:-1]]).astype(np.int32)


def kernel(x, W_emb, W_lin, bias):
  xi = (x - 1) + jnp.asarray(_OFFSETS)[None, :]
  xib = xi.reshape(NW, ROWS_PER_W)
  blk = xib // 8
  off16 = (xib % 8) * EMBED_DIM
  off_fm = (
      off16.reshape(NW, B_PER_W, NUM_FIELDS).transpose(0, 2, 1)
      .reshape(NW, ROWS_PER_W)
  )
  lidx = xi.reshape(NW, B_PER_W, NUM_FIELDS).transpose(0, 2, 1)
  lidx = lidx.reshape(NW, NLC, LCHUNK)
  bias16 = jnp.broadcast_to(bias, (L,)).astype(jnp.float32)
  emb_blk = W_emb.reshape(TOTAL // 8, 128)
  return _fm_call(blk, off_fm, lidx, bias16, emb_blk, W_lin.reshape(-1))
